# baseline (speedup 1.0000x reference)
import functools

import jax
import jax.numpy as jnp
from jax import lax
from jax.experimental import pallas as pl
from jax.experimental.pallas import tpu as pltpu
from jax.experimental.pallas import tpu_sc as plsc

_NB0 = 320
_CH = 128
_SCH = 40


def _sc_gather_sum(features, nidx2, sidx2, B_pad, EMB, K):
    NS = 16
    nb0, CH, SCH = _NB0, _CH, _SCH
    tgt_per_chunk = CH // K
    nc0 = nb0 * K // CH
    n_vec = EMB // 16

    mesh = plsc.VectorSubcoreMesh(core_axis_name="c", subcore_axis_name="s")

    @functools.partial(
        pl.kernel,
        mesh=mesh,
        out_type=(
            jax.ShapeDtypeStruct((B_pad, EMB), jnp.float32),
            jax.ShapeDtypeStruct((B_pad, EMB), jnp.float32),
        ),
        scratch_types=[
            pltpu.VMEM((nc0, CH), jnp.int32),
            pltpu.VMEM((nb0 // SCH, SCH), jnp.int32),
            pltpu.VMEM((CH, EMB), jnp.float32),
            pltpu.VMEM((CH, EMB), jnp.float32),
            pltpu.VMEM((CH, EMB), jnp.float32),
            pltpu.VMEM((CH, EMB), jnp.float32),
            pltpu.VMEM((nb0, EMB), jnp.float32),
            pltpu.SemaphoreType.DMA,
            pltpu.SemaphoreType.DMA,
            pltpu.SemaphoreType.DMA,
            pltpu.SemaphoreType.DMA,
            pltpu.SemaphoreType.DMA,
        ],
    )
    def sc_kernel(feat_hbm, nidx_hbm, sidx_hbm, nsum_hbm, self_hbm,
                  idx_v, sidx_v, buf0, buf1, buf2, buf3, slab,
                  sem0, sem1, sem2, sem3, semw):
        c = lax.axis_index("c")
        s = lax.axis_index("s")
        wid = c * NS + s
        base = wid * nb0
        n_chunks = nc0
        n_sch = nb0 // SCH

        with jax.named_scope("idxload"):
            pltpu.sync_copy(nidx_hbm.at[pl.ds(wid * nc0, nc0)], idx_v)
            pltpu.sync_copy(sidx_hbm.at[pl.ds(wid * n_sch, n_sch)], sidx_v)

        def g_start(ci, buf, sem):
            pltpu.make_async_copy(feat_hbm.at[idx_v.at[ci]], buf, sem).start()

        def g_wait(ci, buf, sem):
            pltpu.make_async_copy(feat_hbm.at[idx_v.at[ci]], buf, sem).wait()

        def accum(buf, ci):
            for bloc in range(tgt_per_chunk):
                def kbody(k4, accs, _bloc=bloc):
                    r0 = _bloc * K + k4 * 4
                    new = accs
                    for kk in range(4):
                        r = r0 + kk
                        new = tuple(new[j] + buf[r, pl.ds(j * 16, 16)]
                                    for j in range(n_vec))
                    return new
                init = tuple(jnp.zeros((16,), jnp.float32) for _ in range(n_vec))
                accs = lax.fori_loop(0, K // 4, kbody, init)
                row = ci * tgt_per_chunk + bloc
                for j in range(n_vec):
                    slab[row, pl.ds(j * 16, 16)] = accs[j]

        bufs = (buf0, buf1, buf2, buf3)
        sems = (sem0, sem1, sem2, sem3)
        nbuf = len(bufs)

        with jax.named_scope("prime"):
            for p in range(nbuf):
                g_start(p, bufs[p], sems[p])

        def body(i, carry):
            cb = nbuf * i
            for p in range(nbuf):
                ci = cb + p
                g_wait(ci, bufs[p], sems[p])
                accum(bufs[p], ci)

                @pl.when(ci + nbuf < n_chunks)
                def _(_p=p, _ci=ci):
                    g_start(_ci + nbuf, bufs[_p], sems[_p])

            return carry

        with jax.named_scope("mainloop"):
            lax.fori_loop(0, n_chunks // nbuf, body, 0)

        with jax.named_scope("tailphase"):
            pltpu.make_async_copy(slab, nsum_hbm.at[pl.ds(base, nb0)], semw).start()
            for j in range(n_sch):
                pltpu.async_copy(feat_hbm.at[sidx_v.at[j]],
                                 bufs[j // 3].at[pl.ds((j % 3) * SCH, SCH)], sem0)
            pltpu.make_async_copy(slab, nsum_hbm.at[pl.ds(base, nb0)], semw).wait()
            for j in range(n_sch):
                pltpu.make_async_copy(feat_hbm.at[sidx_v.at[j]],
                                      bufs[j // 3].at[pl.ds((j % 3) * SCH, SCH)],
                                      sem0).wait()
            for p in range((n_sch + 2) // 3):
                rows = min(3 * SCH, nb0 - p * 3 * SCH)
                pltpu.sync_copy(bufs[p].at[pl.ds(0, rows)],
                                self_hbm.at[pl.ds(base + p * 3 * SCH, rows)])

    return sc_kernel(features, nidx2, sidx2)


def _tc_dense_body(e_ref, t_ref, w1b, w1c, b1r, o_ref):
    es = jnp.sum(e_ref[...], axis=0)
    ts = jnp.sum(t_ref[...], axis=1)
    dn = (((0,), (0,)), ((), ()))
    o_ref[...] = (lax.dot_general(es, w1b[...], dn, preferred_element_type=jnp.float32)
                  + lax.dot_general(ts, w1c[...], dn, preferred_element_type=jnp.float32)
                  + b1r[...])


def _tc_combine_body(ns_ref, sf_ref, pp_ref, w1a, w2a, w2b, b2r, o_ref):
    pre = pp_ref[...] + jnp.dot(ns_ref[...], w1a[...],
                                preferred_element_type=jnp.float32)
    agg = jnp.maximum(pre, 0.0)
    o_ref[...] = (jnp.dot(sf_ref[...], w2a[...], preferred_element_type=jnp.float32)
                  + jnp.dot(agg, w2b[...], preferred_element_type=jnp.float32)
                  + b2r[...])


def kernel(features, neighbor_idx, edge_feats, time_feats, node_idx, W1, b1, W2, b2):
    N, EMB = features.shape
    B, K = neighbor_idx.shape
    EDGE = edge_feats.shape[2]
    TIME = time_feats.shape[2]

    NS = 16
    B_pad = 2 * NS * _NB0
    n_chunks_tot = B_pad * K // _CH

    pad_rows = B_pad - B
    pad_n = (jnp.arange(pad_rows * K, dtype=jnp.int32) % N).reshape(pad_rows, K)
    nidx = jnp.concatenate([neighbor_idx.astype(jnp.int32), pad_n], axis=0)
    nidx2 = nidx.reshape(n_chunks_tot, _CH)
    pad_s = jnp.arange(pad_rows, dtype=jnp.int32) % N
    sidx = jnp.concatenate([node_idx.astype(jnp.int32), pad_s], axis=0)
    sidx2 = sidx.reshape(B_pad // _SCH, _SCH)

    nsum, self_feat = _sc_gather_sum(features, nidx2, sidx2, B_pad, EMB, K)

    W1T = W1.T
    w1a = W1T[:EMB]
    w1b = W1T[EMB:EMB + EDGE]
    w1c = W1T[EMB + EDGE:]
    W2T = W2.T
    w2a = W2T[:EMB]
    w2b = W2T[EMB:]
    b1r = b1.reshape(1, EMB)
    b2r = b2.reshape(1, EMB)

    et = jnp.transpose(edge_feats, (1, 2, 0))
    tt = jnp.transpose(time_feats, (2, 1, 0))

    BLK = 512
    grid = ((B + BLK - 1) // BLK,)
    partial = pl.pallas_call(
        _tc_dense_body,
        grid=grid,
        in_specs=[
            pl.BlockSpec((K, EDGE, BLK), lambda i: (0, 0, i)),
            pl.BlockSpec((TIME, K, BLK), lambda i: (0, 0, i)),
            pl.BlockSpec((EDGE, EMB), lambda i: (0, 0)),
            pl.BlockSpec((TIME, EMB), lambda i: (0, 0)),
            pl.BlockSpec((1, EMB), lambda i: (0, 0)),
        ],
        out_specs=pl.BlockSpec((BLK, EMB), lambda i: (i, 0)),
        out_shape=jax.ShapeDtypeStruct((B, EMB), jnp.float32),
    )(et, tt, w1b, w1c, b1r)

    CBLK = 2048
    cgrid = ((B + CBLK - 1) // CBLK,)
    out = pl.pallas_call(
        _tc_combine_body,
        grid=cgrid,
        in_specs=[
            pl.BlockSpec((CBLK, EMB), lambda i: (i, 0)),
            pl.BlockSpec((CBLK, EMB), lambda i: (i, 0)),
            pl.BlockSpec((CBLK, EMB), lambda i: (i, 0)),
            pl.BlockSpec((EMB, EMB), lambda i: (0, 0)),
            pl.BlockSpec((EMB, EMB), lambda i: (0, 0)),
            pl.BlockSpec((EMB, EMB), lambda i: (0, 0)),
            pl.BlockSpec((1, EMB), lambda i: (0, 0)),
        ],
        out_specs=pl.BlockSpec((CBLK, EMB), lambda i: (i, 0)),
        out_shape=jax.ShapeDtypeStruct((B, EMB), jnp.float32),
    )(nsum, self_feat, partial, w1a, w2a, w2b, b2r)
    return out


# speedup vs baseline: 1.0048x; 1.0048x over previous
import functools

import jax
import jax.numpy as jnp
from jax import lax
from jax.experimental import pallas as pl
from jax.experimental.pallas import tpu as pltpu
from jax.experimental.pallas import tpu_sc as plsc

_NB0 = 320
_CH = 128
_SCH = 40


def _sc_gather_sum(features, nidx2, sidx2, B_pad, EMB, K):
    NS = 16
    nb0, CH, SCH = _NB0, _CH, _SCH
    tgt_per_chunk = CH // K
    nc0 = nb0 * K // CH
    n_vec = EMB // 16

    mesh = plsc.VectorSubcoreMesh(core_axis_name="c", subcore_axis_name="s")

    @functools.partial(
        pl.kernel,
        mesh=mesh,
        out_type=(
            jax.ShapeDtypeStruct((B_pad, EMB), jnp.float32),
            jax.ShapeDtypeStruct((B_pad, EMB), jnp.float32),
        ),
        scratch_types=[
            pltpu.VMEM((nc0, CH), jnp.int32),
            pltpu.VMEM((nb0 // SCH, SCH), jnp.int32),
            pltpu.VMEM((CH, EMB), jnp.float32),
            pltpu.VMEM((CH, EMB), jnp.float32),
            pltpu.VMEM((CH, EMB), jnp.float32),
            pltpu.VMEM((CH, EMB), jnp.float32),
            pltpu.VMEM((nb0, EMB), jnp.float32),
            pltpu.SemaphoreType.DMA,
            pltpu.SemaphoreType.DMA,
            pltpu.SemaphoreType.DMA,
            pltpu.SemaphoreType.DMA,
            pltpu.SemaphoreType.DMA,
        ],
    )
    def sc_kernel(feat_hbm, nidx_hbm, sidx_hbm, nsum_hbm, self_hbm,
                  idx_v, sidx_v, buf0, buf1, buf2, buf3, slab,
                  sem0, sem1, sem2, sem3, semw):
        c = lax.axis_index("c")
        s = lax.axis_index("s")
        wid = c * NS + s
        base = wid * nb0
        n_chunks = nc0
        n_sch = nb0 // SCH

        with jax.named_scope("idxload"):
            pltpu.sync_copy(nidx_hbm.at[pl.ds(wid * nc0, nc0)], idx_v)
            pltpu.sync_copy(sidx_hbm.at[pl.ds(wid * n_sch, n_sch)], sidx_v)

        def g_start(ci, buf, sem):
            pltpu.make_async_copy(feat_hbm.at[idx_v.at[ci]], buf, sem).start()

        def g_wait(ci, buf, sem):
            pltpu.make_async_copy(feat_hbm.at[idx_v.at[ci]], buf, sem).wait()

        def accum(buf, ci):
            for bloc in range(tgt_per_chunk):
                def kbody(k4, accs, _bloc=bloc):
                    r0 = _bloc * K + k4 * 4
                    new = accs
                    for kk in range(4):
                        r = r0 + kk
                        new = tuple(new[j] + buf[r, pl.ds(j * 16, 16)]
                                    for j in range(n_vec))
                    return new
                init = tuple(jnp.zeros((16,), jnp.float32) for _ in range(n_vec))
                accs = lax.fori_loop(0, K // 4, kbody, init)
                row = ci * tgt_per_chunk + bloc
                for j in range(n_vec):
                    slab[row, pl.ds(j * 16, 16)] = accs[j]

        bufs = (buf0, buf1, buf2, buf3)
        sems = (sem0, sem1, sem2, sem3)
        nbuf = len(bufs)

        with jax.named_scope("prime"):
            for p in range(nbuf):
                g_start(p, bufs[p], sems[p])

        def body(i, carry):
            cb = nbuf * i
            for p in range(nbuf):
                ci = cb + p
                g_wait(ci, bufs[p], sems[p])
                accum(bufs[p], ci)

                @pl.when(ci + nbuf < n_chunks)
                def _(_p=p, _ci=ci):
                    g_start(_ci + nbuf, bufs[_p], sems[_p])

            return carry

        with jax.named_scope("mainloop"):
            lax.fori_loop(0, n_chunks // nbuf, body, 0)

        with jax.named_scope("tailphase"):
            pltpu.make_async_copy(slab, nsum_hbm.at[pl.ds(base, nb0)], semw).start()
            for j in range(n_sch):
                pltpu.async_copy(feat_hbm.at[sidx_v.at[j]],
                                 bufs[j // 3].at[pl.ds((j % 3) * SCH, SCH)], sem0)
            pltpu.make_async_copy(slab, nsum_hbm.at[pl.ds(base, nb0)], semw).wait()
            for j in range(n_sch):
                pltpu.make_async_copy(feat_hbm.at[sidx_v.at[j]],
                                      bufs[j // 3].at[pl.ds((j % 3) * SCH, SCH)],
                                      sem0).wait()
            for p in range((n_sch + 2) // 3):
                rows = min(3 * SCH, nb0 - p * 3 * SCH)
                pltpu.sync_copy(bufs[p].at[pl.ds(0, rows)],
                                self_hbm.at[pl.ds(base + p * 3 * SCH, rows)])

    return sc_kernel(features, nidx2, sidx2)


def _tc_dense_body(e_ref, t_ref, w1b, w1c, b1r, o_ref):
    es = jnp.sum(e_ref[...], axis=0)
    ts = jnp.sum(t_ref[...], axis=1)
    dn = (((0,), (0,)), ((), ()))
    o_ref[...] = (lax.dot_general(es, w1b[...], dn, preferred_element_type=jnp.float32)
                  + lax.dot_general(ts, w1c[...], dn, preferred_element_type=jnp.float32)
                  + b1r[...])


def _tc_combine_body(ns_ref, sf_ref, pp_ref, w1a, w2a, w2b, b2r, o_ref):
    pre = pp_ref[...] + jnp.dot(ns_ref[...], w1a[...],
                                preferred_element_type=jnp.float32)
    agg = jnp.maximum(pre, 0.0)
    o_ref[...] = (jnp.dot(sf_ref[...], w2a[...], preferred_element_type=jnp.float32)
                  + jnp.dot(agg, w2b[...], preferred_element_type=jnp.float32)
                  + b2r[...])


def kernel(features, neighbor_idx, edge_feats, time_feats, node_idx, W1, b1, W2, b2):
    N, EMB = features.shape
    B, K = neighbor_idx.shape
    EDGE = edge_feats.shape[2]
    TIME = time_feats.shape[2]

    NS = 16
    B_pad = 2 * NS * _NB0
    n_chunks_tot = B_pad * K // _CH

    pad_rows = B_pad - B
    pad_n = (jnp.arange(pad_rows * K, dtype=jnp.int32) % N).reshape(pad_rows, K)
    nidx = jnp.concatenate([neighbor_idx.astype(jnp.int32), pad_n], axis=0)
    nidx2 = nidx.reshape(n_chunks_tot, _CH)
    pad_s = jnp.arange(pad_rows, dtype=jnp.int32) % N
    sidx = jnp.concatenate([node_idx.astype(jnp.int32), pad_s], axis=0)
    sidx2 = sidx.reshape(B_pad // _SCH, _SCH)

    nsum, self_feat = _sc_gather_sum(features, nidx2, sidx2, B_pad, EMB, K)

    W1T = W1.T
    w1a = W1T[:EMB]
    w1b = W1T[EMB:EMB + EDGE]
    w1c = W1T[EMB + EDGE:]
    W2T = W2.T
    w2a = W2T[:EMB]
    w2b = W2T[EMB:]
    b1r = b1.reshape(1, EMB)
    b2r = b2.reshape(1, EMB)

    et = jnp.transpose(edge_feats, (1, 2, 0))
    tt = jnp.transpose(time_feats, (2, 1, 0))

    BLK = 512
    grid = ((B + BLK - 1) // BLK,)
    partial = pl.pallas_call(
        _tc_dense_body,
        grid=grid,
        in_specs=[
            pl.BlockSpec((K, EDGE, BLK), lambda i: (0, 0, i)),
            pl.BlockSpec((TIME, K, BLK), lambda i: (0, 0, i)),
            pl.BlockSpec((EDGE, EMB), lambda i: (0, 0)),
            pl.BlockSpec((TIME, EMB), lambda i: (0, 0)),
            pl.BlockSpec((1, EMB), lambda i: (0, 0)),
        ],
        out_specs=pl.BlockSpec((BLK, EMB), lambda i: (i, 0)),
        out_shape=jax.ShapeDtypeStruct((B, EMB), jnp.float32),
    )(et, tt, w1b, w1c, b1r)

    CBLK = 4096
    cgrid = ((B + CBLK - 1) // CBLK,)
    out = pl.pallas_call(
        _tc_combine_body,
        grid=cgrid,
        in_specs=[
            pl.BlockSpec((CBLK, EMB), lambda i: (i, 0)),
            pl.BlockSpec((CBLK, EMB), lambda i: (i, 0)),
            pl.BlockSpec((CBLK, EMB), lambda i: (i, 0)),
            pl.BlockSpec((EMB, EMB), lambda i: (0, 0)),
            pl.BlockSpec((EMB, EMB), lambda i: (0, 0)),
            pl.BlockSpec((EMB, EMB), lambda i: (0, 0)),
            pl.BlockSpec((1, EMB), lambda i: (0, 0)),
        ],
        out_specs=pl.BlockSpec((CBLK, EMB), lambda i: (i, 0)),
        out_shape=jax.ShapeDtypeStruct((B, EMB), jnp.float32),
    )(nsum, self_feat, partial, w1a, w2a, w2b, b2r)
    return out
